# baseline (speedup 1.0000x reference)
import functools

import jax
import jax.numpy as jnp
from jax.experimental import pallas as pl
from jax.experimental.pallas import tpu as pltpu

B, D = 16384, 64
D2 = 2 * D
TILE = 2048


def _fused_kernel(xh_ref, v_ref, bias_ref, g_ref, b_ref, o_ref):
    vd = jnp.concatenate([v_ref[...], v_ref[...]], axis=1)
    pack = lambda r: jnp.concatenate([r[...], r[...]], axis=1)
    bias2 = pack(bias_ref)
    g2 = pack(g_ref)
    b2 = pack(b_ref)
    ri = jax.lax.broadcasted_iota(jnp.int32, (D2, D2), 0) // D
    ci = jax.lax.broadcasted_iota(jnp.int32, (D2, D2), 1) // D
    avgd = jnp.where(ri == ci, 1.0 / D, 0.0).astype(jnp.float32)

    pre = jnp.dot(xh_ref[...], vd, preferred_element_type=jnp.float32) + bias2
    mu = jnp.dot(pre, avgd, preferred_element_type=jnp.float32)
    ex2 = jnp.dot(pre * pre, avgd, preferred_element_type=jnp.float32)
    var = ex2 - mu * mu
    normed = (pre - mu) * jax.lax.rsqrt(var + 1e-5) * g2 + b2
    o_ref[...] = jax.nn.sigmoid(normed)


@functools.partial(jax.jit, static_argnames=("interpret",))
def _run(x, h_prev, W_w, U_w, bias, ln_g, ln_b, interpret=False):
    xh = jnp.concatenate([x, h_prev], axis=1)
    v = jnp.concatenate([W_w, U_w], axis=0)
    grid = (B // TILE,)
    row_spec = pl.BlockSpec((TILE, D2), lambda i: (i, 0))
    wide = pl.pallas_call(
        _fused_kernel,
        grid=grid,
        in_specs=[row_spec,
                  pl.BlockSpec((D2, D), lambda i: (0, 0)),
                  pl.BlockSpec((1, D), lambda i: (0, 0)),
                  pl.BlockSpec((1, D), lambda i: (0, 0)),
                  pl.BlockSpec((1, D), lambda i: (0, 0))],
        out_specs=row_spec,
        out_shape=jax.ShapeDtypeStruct((B, D2), jnp.float32),
        compiler_params=pltpu.CompilerParams(
            dimension_semantics=("parallel",),
            allow_input_fusion=[True, False, False, False, False],
        ),
        interpret=interpret,
    )(xh, v, bias, ln_g, ln_b)
    return wide[:, :D]


def kernel(x, h_prev, W_w, W_b, U_w, U_b, M_w, M_b, Qr_w, Qr_b, Ql_w, Ql_b, ln_g, ln_b):
    r = lambda v: v.reshape(1, D)
    bias = (W_b + U_b + Qr_b + Ql_b).reshape(1, D)
    return _run(x, h_prev, W_w, U_w, bias, r(ln_g), r(ln_b))


# speedup vs baseline: 1.0680x; 1.0680x over previous
import functools

import jax
import jax.numpy as jnp
from jax.experimental import pallas as pl
from jax.experimental.pallas import tpu as pltpu

B, D = 16384, 64
TILE = 2048


def _fused_kernel(x_ref, h_ref, w_ref, u_ref, wb_ref, ub_ref, qrb_ref, qlb_ref,
                  g_ref, b_ref, o_ref):
    pre = jnp.dot(x_ref[...], w_ref[...], preferred_element_type=jnp.float32)
    pre = pre + jnp.dot(h_ref[...], u_ref[...], preferred_element_type=jnp.float32)
    pre = pre + (wb_ref[...] + ub_ref[...] + qrb_ref[...] + qlb_ref[...])
    mu = jnp.mean(pre, axis=-1, keepdims=True)
    cent = pre - mu
    var = jnp.mean(cent * cent, axis=-1, keepdims=True)
    normed = cent * jax.lax.rsqrt(var + 1e-5) * g_ref[...] + b_ref[...]
    o_ref[...] = jax.nn.sigmoid(normed)


@functools.partial(jax.jit, static_argnames=("interpret",))
def _run(x, h_prev, W_w, U_w, W_b, U_b, Qr_b, Ql_b, ln_g, ln_b, interpret=False):
    grid = (B // TILE,)
    row_spec = pl.BlockSpec((TILE, D), lambda i: (i, 0))
    full_spec = pl.BlockSpec((D, D), lambda i: (0, 0))
    vec_spec = pl.BlockSpec((1, D), lambda i: (0, 0))
    return pl.pallas_call(
        _fused_kernel,
        grid=grid,
        in_specs=[row_spec, row_spec, full_spec, full_spec,
                  vec_spec, vec_spec, vec_spec, vec_spec, vec_spec, vec_spec],
        out_specs=row_spec,
        out_shape=jax.ShapeDtypeStruct((B, D), jnp.float32),
        compiler_params=pltpu.CompilerParams(dimension_semantics=("parallel",)),
        interpret=interpret,
    )(x, h_prev, W_w, U_w, W_b, U_b, Qr_b, Ql_b, ln_g, ln_b)


def kernel(x, h_prev, W_w, W_b, U_w, U_b, M_w, M_b, Qr_w, Qr_b, Ql_w, Ql_b, ln_g, ln_b):
    r = lambda v: v.reshape(1, D)
    return _run(x, h_prev, W_w, U_w, r(W_b), r(U_b), r(Qr_b), r(Ql_b), r(ln_g), r(ln_b))
